# Initial kernel scaffold; baseline (speedup 1.0000x reference)
import functools

import jax
import jax.numpy as jnp
from jax import lax
from jax.experimental import pallas as pl
from jax.experimental.pallas import tpu as pltpu
from jax.experimental.pallas import tpu_sc as plsc

B = 64
H = 2048
D = 128
NC = 2
NS = 16
NW = NC * NS
BPW = B // NW
E = BPW * H
GCH = 128
NG = E // GCH


def _sc_body(pred_hbm, targ_hbm, out_hbm, idx_v, col_v, pr_v, pp_v, tr_v,
             tp_v, res_v, sem):
    cid = lax.axis_index("c")
    sid = lax.axis_index("s")
    wid = cid * NS + sid
    b0 = wid * BPW
    lanes = lax.iota(jnp.int32, 16)

    base = b0 * H * D + 1

    def build(c, _):
        idx_v[pl.ds(c * 16, 16)] = base + (c * 16 + lanes) * D
        return 0

    lax.fori_loop(0, E // 16, build, 0)

    def fire(i, _):
        pltpu.async_copy(
            targ_hbm.at[idx_v.at[pl.ds(i * GCH, GCH)]],
            col_v.at[pl.ds(i * GCH, GCH)], sem)
        return 0

    lax.fori_loop(0, NG, fire, 0)

    def drain(i, _):
        pltpu.make_async_copy(
            targ_hbm.at[idx_v.at[pl.ds(i * GCH, GCH)]],
            col_v.at[pl.ds(i * GCH, GCH)], sem).wait()
        return 0

    lax.fori_loop(0, NG, drain, 0)

    total = jnp.float32(0.0)
    for k in range(BPW):
        def cnt(c, acc):
            v = col_v[pl.ds(k * H + c * 16, 16)]
            return acc + jnp.where(v != 0.0, 1, 0).astype(jnp.int32)

        accv = lax.fori_loop(0, H // 16, cnt, jnp.zeros((16,), jnp.int32))
        t = jnp.sum(accv)

        r = jnp.where(t == 0, H - 1, t - 1)
        rp = jnp.maximum(r - 1, 0)
        bb = b0 + k
        g = (bb * H + r) * D
        gp = (bb * H + rp) * D
        pltpu.sync_copy(pred_hbm.at[pl.ds(g, D)], pr_v)
        pltpu.sync_copy(targ_hbm.at[pl.ds(g, D)], tr_v)
        pltpu.sync_copy(pred_hbm.at[pl.ds(gp, D)], pp_v)
        pltpu.sync_copy(targ_hbm.at[pl.ds(gp, D)], tp_v)

        x = t.astype(jnp.float32) / jnp.float32(H - 1)
        xv = jnp.full((16,), x, jnp.float32)
        iv = plsc.bitcast(xv, jnp.int32)
        yv = plsc.bitcast(jnp.int32(0x5F3759DF) - (iv >> 1), jnp.float32)
        for _ in range(3):
            yv = yv * (1.5 - 0.5 * xv * yv * yv)
        pwv = xv * xv * (xv * yv)
        pw = jnp.sum(jnp.where(lanes == 0, pwv, 0.0))
        pw = jnp.where(t == 0, 0.0, pw)
        wgt = 1.0 + 0.5 * pw

        pm = jnp.where(r > 0, 1.0, 0.0)

        def lossstep(c, carry):
            sq, l1 = carry
            dd = pr_v[pl.ds(c * 16, 16)] - tr_v[pl.ds(c * 16, 16)]
            dp = (pp_v[pl.ds(c * 16, 16)] - tp_v[pl.ds(c * 16, 16)]) * pm
            return (sq + dd * dd, l1 + jnp.abs(dd - dp))

        z = jnp.zeros((16,), jnp.float32)
        sq, l1 = lax.fori_loop(0, D // 16, lossstep, (z, z))
        total = total + wgt * (jnp.sum(sq) + jnp.sum(l1))

    res_v[...] = jnp.where(lanes == 0, jnp.full((16,), total, jnp.float32),
                           jnp.zeros((16,), jnp.float32))
    pltpu.sync_copy(res_v, out_hbm.at[wid])


@jax.jit
def _sc_loss(pred_flat, targ_flat):
    mesh = plsc.VectorSubcoreMesh(core_axis_name="c", subcore_axis_name="s")
    f = pl.kernel(
        _sc_body,
        out_type=jax.ShapeDtypeStruct((NW, 16), jnp.float32),
        mesh=mesh,
        scratch_types=[
            pltpu.VMEM((E,), jnp.int32),
            pltpu.VMEM((E,), jnp.float32),
            pltpu.VMEM((D,), jnp.float32),
            pltpu.VMEM((D,), jnp.float32),
            pltpu.VMEM((D,), jnp.float32),
            pltpu.VMEM((D,), jnp.float32),
            pltpu.VMEM((16,), jnp.float32),
            pltpu.SemaphoreType.DMA,
        ],
    )
    return f(pred_flat, targ_flat)


def kernel(pred, targ):
    out = _sc_loss(pred.reshape(-1), targ.reshape(-1))
    loss = jnp.sum(out) * (1.0 / B)
    return (loss, {"a0_loss": loss})


# speedup vs baseline: 4.3461x; 4.3461x over previous
import functools

import jax
import jax.numpy as jnp
from jax import lax
from jax.experimental import pallas as pl
from jax.experimental.pallas import tpu as pltpu
from jax.experimental.pallas import tpu_sc as plsc

B = 64
H = 2048
D = 128
NC = 2
NS = 16
NW = NC * NS
BPW = B // NW
E = BPW * H
GCH = 128
NG = E // GCH


def _sc_body(pred_hbm, targ_hbm, out_hbm, idx_v, col_v, pr_v, pp_v, tr_v,
             tp_v, res_v, sem):
    cid = lax.axis_index("c")
    sid = lax.axis_index("s")
    wid = cid * NS + sid
    b0 = wid * BPW
    lanes = lax.iota(jnp.int32, 16)

    base = b0 * H * D + 1

    def build(c, _):
        idx_v[pl.ds(c * 16, 16)] = base + (c * 16 + lanes) * D
        return 0

    lax.fori_loop(0, E // 16, build, 0)

    def fire(i, _):
        pltpu.async_copy(
            targ_hbm.at[idx_v.at[pl.ds(i * GCH, GCH)]],
            col_v.at[pl.ds(i * GCH, GCH)], sem)
        return 0

    lax.fori_loop(0, NG, fire, 0)

    def drain(i, _):
        pltpu.make_async_copy(
            targ_hbm.at[idx_v.at[pl.ds(i * GCH, GCH)]],
            col_v.at[pl.ds(i * GCH, GCH)], sem).wait()
        return 0

    lax.fori_loop(0, NG, drain, 0)

    total = jnp.float32(0.0)
    for k in range(BPW):
        def cnt(c, acc):
            v = col_v[pl.ds(k * H + c * 16, 16)]
            return acc + jnp.where(v != 0.0, 1, 0).astype(jnp.int32)

        accv = lax.fori_loop(0, H // 16, cnt, jnp.zeros((16,), jnp.int32))
        t = jnp.sum(accv)

        r = jnp.where(t == 0, H - 1, t - 1)
        rp = jnp.maximum(r - 1, 0)
        bb = b0 + k
        g = (bb * H + r) * D
        gp = (bb * H + rp) * D
        pltpu.sync_copy(pred_hbm.at[pl.ds(g, D)], pr_v)
        pltpu.sync_copy(targ_hbm.at[pl.ds(g, D)], tr_v)
        pltpu.sync_copy(pred_hbm.at[pl.ds(gp, D)], pp_v)
        pltpu.sync_copy(targ_hbm.at[pl.ds(gp, D)], tp_v)

        x = t.astype(jnp.float32) * jnp.float32(1.0 / (H - 1))
        xv = jnp.full((16,), x, jnp.float32)
        iv = plsc.bitcast(xv, jnp.int32)
        yv = plsc.bitcast(jnp.int32(0x5F3759DF) - (iv >> 1), jnp.float32)
        for _ in range(3):
            yv = yv * (1.5 - 0.5 * xv * yv * yv)
        pwv = xv * xv * (xv * yv)
        pw = jnp.sum(jnp.where(lanes == 0, pwv, 0.0))
        pw = jnp.where(t == 0, 0.0, pw)
        wgt = 1.0 + 0.5 * pw

        pm = jnp.where(r > 0, 1.0, 0.0)

        def lossstep(c, carry):
            sq, l1 = carry
            dd = pr_v[pl.ds(c * 16, 16)] - tr_v[pl.ds(c * 16, 16)]
            dp = (pp_v[pl.ds(c * 16, 16)] - tp_v[pl.ds(c * 16, 16)]) * pm
            return (sq + dd * dd, l1 + jnp.abs(dd - dp))

        z = jnp.zeros((16,), jnp.float32)
        sq, l1 = lax.fori_loop(0, D // 16, lossstep, (z, z))
        total = total + wgt * (jnp.sum(sq) + jnp.sum(l1))

    res_v[...] = jnp.where(lanes == 0, jnp.full((16,), total, jnp.float32),
                           jnp.zeros((16,), jnp.float32))
    pltpu.sync_copy(res_v, out_hbm.at[wid])


@jax.jit
def _sc_loss(pred_flat, targ_flat):
    mesh = plsc.VectorSubcoreMesh(core_axis_name="c", subcore_axis_name="s")
    f = pl.kernel(
        _sc_body,
        out_type=jax.ShapeDtypeStruct((NW, 16), jnp.float32),
        mesh=mesh,
        compiler_params=pltpu.CompilerParams(needs_layout_passes=False),
        scratch_types=[
            pltpu.VMEM((E,), jnp.int32),
            pltpu.VMEM((E,), jnp.float32),
            pltpu.VMEM((D,), jnp.float32),
            pltpu.VMEM((D,), jnp.float32),
            pltpu.VMEM((D,), jnp.float32),
            pltpu.VMEM((D,), jnp.float32),
            pltpu.VMEM((16,), jnp.float32),
            pltpu.SemaphoreType.DMA,
        ],
    )
    return f(pred_flat, targ_flat)


def kernel(pred, targ):
    out = _sc_loss(pred.reshape(-1), targ.reshape(-1))
    loss = jnp.sum(out) * (1.0 / B)
    return (loss, {"a0_loss": loss})


# speedup vs baseline: 4.9669x; 1.1428x over previous
import functools

import jax
import jax.numpy as jnp
from jax import lax
from jax.experimental import pallas as pl
from jax.experimental.pallas import tpu as pltpu
from jax.experimental.pallas import tpu_sc as plsc

B = 64
H = 2048
D = 128
NC = 2
NS = 16
NW = NC * NS
BPW = B // NW
E = BPW * H
GCH = 128
NG = E // GCH


def _sc_body(pred_hbm, targ_hbm, idx_hbm, out_hbm, idx_v, col_v, row_v, res_v,
             sem, sem2):
    cid = lax.axis_index("c")
    sid = lax.axis_index("s")
    wid = cid * NS + sid
    b0 = wid * BPW
    lanes = lax.iota(jnp.int32, 16)

    pltpu.sync_copy(idx_hbm.at[pl.ds(b0 * H, E)], idx_v)

    def chunk_fire(c, _):
        off = c * GCH
        pltpu.async_copy(
            targ_hbm.at[idx_v.at[pl.ds(off, GCH)]],
            col_v.at[pl.ds(off, GCH)], sem)
        return 0

    lax.fori_loop(0, NG, chunk_fire, 0)

    CPB = H // GCH

    def count_batch(k):
        def cnt(c, acc):
            off = (k * CPB + c) * GCH
            pltpu.make_async_copy(
                targ_hbm.at[idx_v.at[pl.ds(off, GCH)]],
                col_v.at[pl.ds(off, GCH)], sem).wait()
            for u in range(GCH // 16):
                v = col_v[pl.ds(off + u * 16, 16)]
                acc = acc + jnp.where(v != 0.0, 1, 0).astype(jnp.int32)
            return acc

        accv = lax.fori_loop(0, CPB, cnt, jnp.zeros((16,), jnp.int32))
        return jnp.sum(accv)

    ts = []
    rs = []
    for k in range(BPW):
        t = count_batch(k)
        r = jnp.where(t == 0, H - 1, t - 1)
        rp = jnp.maximum(r - 1, 0)
        bb = b0 + k
        g = (bb * H + r) * D
        gp = (bb * H + rp) * D
        o = k * 4 * D
        pltpu.async_copy(pred_hbm.at[pl.ds(g, D)], row_v.at[pl.ds(o, D)], sem2)
        pltpu.async_copy(targ_hbm.at[pl.ds(g, D)],
                         row_v.at[pl.ds(o + D, D)], sem2)
        pltpu.async_copy(pred_hbm.at[pl.ds(gp, D)],
                         row_v.at[pl.ds(o + 2 * D, D)], sem2)
        pltpu.async_copy(targ_hbm.at[pl.ds(gp, D)],
                         row_v.at[pl.ds(o + 3 * D, D)], sem2)
        ts.append(t)
        rs.append(r)

    for k in range(BPW):
        o = k * 4 * D
        g = (b0 + k) * H * D
        pltpu.make_async_copy(pred_hbm.at[pl.ds(g, D)],
                              row_v.at[pl.ds(o, D)], sem2).wait()
        pltpu.make_async_copy(pred_hbm.at[pl.ds(g, D)],
                              row_v.at[pl.ds(o + D, D)], sem2).wait()
        pltpu.make_async_copy(pred_hbm.at[pl.ds(g, D)],
                              row_v.at[pl.ds(o + 2 * D, D)], sem2).wait()
        pltpu.make_async_copy(pred_hbm.at[pl.ds(g, D)],
                              row_v.at[pl.ds(o + 3 * D, D)], sem2).wait()

    total = jnp.float32(0.0)
    for k in range(BPW):
        t, r = ts[k], rs[k]
        x = t.astype(jnp.float32) * jnp.float32(1.0 / (H - 1))
        xv = jnp.full((16,), x, jnp.float32)
        iv = plsc.bitcast(xv, jnp.int32)
        yv = plsc.bitcast(jnp.int32(0x5F3759DF) - (iv >> 1), jnp.float32)
        for _ in range(3):
            yv = yv * (1.5 - 0.5 * xv * yv * yv)
        pwv = xv * xv * (xv * yv)
        pw = jnp.sum(jnp.where(lanes == 0, pwv, 0.0))
        pw = jnp.where(t == 0, 0.0, pw)
        wgt = 1.0 + 0.5 * pw

        pm = jnp.where(r > 0, 1.0, 0.0)
        o = k * 4 * D
        z = jnp.zeros((16,), jnp.float32)
        sq, l1 = z, z
        for c in range(D // 16):
            dd = (row_v[pl.ds(o + c * 16, 16)]
                  - row_v[pl.ds(o + D + c * 16, 16)])
            dp = (row_v[pl.ds(o + 2 * D + c * 16, 16)]
                  - row_v[pl.ds(o + 3 * D + c * 16, 16)]) * pm
            sq = sq + dd * dd
            l1 = l1 + jnp.abs(dd - dp)
        total = total + wgt * (jnp.sum(sq) + jnp.sum(l1))

    res_v[...] = jnp.where(lanes == 0, jnp.full((16,), total, jnp.float32),
                           jnp.zeros((16,), jnp.float32))
    pltpu.sync_copy(res_v, out_hbm.at[wid])


@jax.jit
def _sc_loss(pred_flat, targ_flat, idx_tab):
    mesh = plsc.VectorSubcoreMesh(core_axis_name="c", subcore_axis_name="s")
    f = pl.kernel(
        _sc_body,
        out_type=jax.ShapeDtypeStruct((NW, 16), jnp.float32),
        mesh=mesh,
        compiler_params=pltpu.CompilerParams(needs_layout_passes=False),
        scratch_types=[
            pltpu.VMEM((E,), jnp.int32),
            pltpu.VMEM((E,), jnp.float32),
            pltpu.VMEM((4 * BPW * D,), jnp.float32),
            pltpu.VMEM((16,), jnp.float32),
            pltpu.SemaphoreType.DMA,
            pltpu.SemaphoreType.DMA,
        ],
    )
    return f(pred_flat, targ_flat, idx_tab)


def kernel(pred, targ):
    idx_tab = jnp.arange(B * H, dtype=jnp.int32) * D + 1
    out = _sc_loss(pred.reshape(-1), targ.reshape(-1), idx_tab)
    loss = jnp.sum(out) * (1.0 / B)
    return (loss, {"a0_loss": loss})
